# baseline (speedup 1.0000x reference)
import functools

import numpy as np
import jax
import jax.numpy as jnp
from jax import lax
from jax.experimental import pallas as pl
from jax.experimental.pallas import tpu as pltpu
from jax.experimental.pallas import tpu_sc as plsc

N = 10000
F = 128
E_EDGES = 320000
NB = 1024

NC = 2
NS = 16
L = 16
NW = NC * NS

CH = 80
ROW_CHUNKS = N // CH
EDGE_PER_TILE = E_EDGES // NS
EDGE_CHUNKS = EDGE_PER_TILE // CH


_PE_DIV = np.exp(
    -np.floor(np.arange(F, dtype=np.float64) / 2.0)
    * (np.log(10000.0) / (F // 2))
).astype(np.float32)

_TC_BLK = 1000


def _tc_stage1(x_u, x_i, batch_user, seed_time, time_user,
               W_enc_u, b_enc_u, W_enc_i, b_enc_i,
               W_time, b_time, W1_ui_l, W1_iu_l, W1_iu_r, b1_u,
               W1_ui_r, b1_i):
    blk = _TC_BLK

    def body(xu_ref, xi_ref, bat_ref, seed_ref, tim_ref, pediv,
             weu, beu, wei, bei, wt, bt,
             wl_ui, wl_iu, wr_iu, bu, wr_ui, bi,
             gu_ref, gi_ref, ru_ref, ri_ref):
        dot = functools.partial(jnp.dot, preferred_element_type=jnp.float32)
        bid = lax.broadcasted_iota(jnp.int32, (blk, NB), 1)
        onehot = (bid == bat_ref[...]).astype(jnp.float32)
        rel = jnp.dot(onehot, seed_ref[...],
                      preferred_element_type=jnp.float32,
                      precision=lax.Precision.HIGHEST) - tim_ref[...]
        arg = rel * pediv[...]
        lane = lax.broadcasted_iota(jnp.int32, (blk, F), 1)
        pe = jnp.where(lane % 2 == 0, jnp.sin(arg), jnp.cos(arg))
        hu = dot(xu_ref[...], weu[...]) + beu[...] + dot(pe, wt[...]) + bt[...]
        hi = dot(xi_ref[...], wei[...]) + bei[...]
        gu_ref[...] = dot(hu, wl_ui[...])
        gi_ref[...] = dot(hi, wl_iu[...])
        ru_ref[...] = dot(hu, wr_iu[...]) + bu[...]
        ri_ref[...] = dot(hi, wr_ui[...]) + bi[...]

    row_spec = pl.BlockSpec((blk, F), lambda i: (i, 0))
    one_spec = pl.BlockSpec((blk, 1), lambda i: (i, 0))
    seed_spec = pl.BlockSpec((NB, 1), lambda i: (0, 0))
    w_spec = pl.BlockSpec((F, F), lambda i: (0, 0))
    b_spec = pl.BlockSpec((1, F), lambda i: (0, 0))
    out_sds = jax.ShapeDtypeStruct((N, F), jnp.float32)
    return pl.pallas_call(
        body,
        grid=(N // blk,),
        in_specs=[row_spec, row_spec, one_spec, seed_spec, one_spec, b_spec,
                  w_spec, b_spec, w_spec, b_spec, w_spec, b_spec,
                  w_spec, w_spec, w_spec, b_spec, w_spec, b_spec],
        out_specs=[row_spec, row_spec, row_spec, row_spec],
        out_shape=[out_sds, out_sds, out_sds, out_sds],
    )(x_u, x_i, batch_user.reshape(N, 1), seed_time.reshape(NB, 1),
      time_user.reshape(N, 1), jnp.asarray(_PE_DIV).reshape(1, F),
      W_enc_u, b_enc_u.reshape(1, F), W_enc_i, b_enc_i.reshape(1, F),
      W_time, b_time.reshape(1, F),
      W1_ui_l, W1_iu_l, W1_iu_r, b1_u.reshape(1, F),
      W1_ui_r, b1_i.reshape(1, F))


def _tc_stage2(msg_u, r_u, msg_i, r_i, W2_ui_l, W2_iu_l, W2_iu_r, b2_u,
               W2_ui_r, b2_i):
    blk = _TC_BLK

    def body(mu_ref, ru_ref, mi_ref, ri_ref,
             wl_ui, wl_iu, wr_iu, bu, wr_ui, bi,
             g2u_ref, g2i_ref, r2u_ref, r2i_ref):
        u1 = jnp.maximum(mu_ref[...] + ru_ref[...], 0.0)
        i1 = jnp.maximum(mi_ref[...] + ri_ref[...], 0.0)
        dot = functools.partial(jnp.dot, preferred_element_type=jnp.float32)
        g2u_ref[...] = dot(u1, wl_ui[...])
        g2i_ref[...] = dot(i1, wl_iu[...])
        r2u_ref[...] = dot(u1, wr_iu[...]) + bu[...]
        r2i_ref[...] = dot(i1, wr_ui[...]) + bi[...]

    row_spec = pl.BlockSpec((blk, F), lambda i: (i, 0))
    w_spec = pl.BlockSpec((F, F), lambda i: (0, 0))
    b_spec = pl.BlockSpec((1, F), lambda i: (0, 0))
    out_sds = jax.ShapeDtypeStruct((N, F), jnp.float32)
    return pl.pallas_call(
        body,
        grid=(N // blk,),
        in_specs=[row_spec, row_spec, row_spec, row_spec,
                  w_spec, w_spec, w_spec, b_spec, w_spec, b_spec],
        out_specs=[row_spec, row_spec, row_spec, row_spec],
        out_shape=[out_sds, out_sds, out_sds, out_sds],
    )(msg_u, r_u, msg_i, r_i,
      W2_ui_l, W2_iu_l, W2_iu_r, b2_u.reshape(1, F),
      W2_ui_r, b2_i.reshape(1, F))


IB = 25
NBLK = EDGE_CHUNKS // IB
NBUF = 4


def _segsum_pair(table_u, table_i, ui_src, ui_dst, iu_src, iu_dst,
                 res_u, res_i, add_residual):
    mesh = plsc.VectorSubcoreMesh(core_axis_name="c", subcore_axis_name="s")

    @functools.partial(
        pl.kernel, mesh=mesh,
        out_type=(jax.ShapeDtypeStruct((N, F), jnp.float32),
                  jax.ShapeDtypeStruct((N, F), jnp.float32)),
        scratch_types=[
            pltpu.VMEM_SHARED((N, F), jnp.float32),
            pltpu.VMEM((IB * CH,), jnp.int32),
        ]
        + [pltpu.VMEM((CH,), jnp.int32) for _ in range(NBUF)]
        + [pltpu.VMEM((CH, F), jnp.float32) for _ in range(NBUF)]
        + [pltpu.SemaphoreType.DMA for _ in range(3 * NBUF)],
    )
    def k(tu_hbm, ti_hbm, ui_s_hbm, ui_d_hbm, iu_s_hbm, iu_d_hbm,
          ru_hbm, ri_hbm, out_u_hbm, out_i_hbm,
          acc, idx_sv, *bufs_and_sems):
        cid = lax.axis_index("c")
        sid = lax.axis_index("s")
        dbufs = bufs_and_sems[0:NBUF]
        bufs = bufs_and_sems[NBUF:2 * NBUF]
        sems = bufs_and_sems[2 * NBUF:3 * NBUF]
        dsems = bufs_and_sems[3 * NBUF:4 * NBUF]
        ssems = bufs_and_sems[4 * NBUF:]
        rows = bufs[0]
        resb = bufs[1]

        def zrow(r, carry):
            for j in range(F // L):
                rows[r, pl.ds(j * L, L)] = jnp.zeros((L,), jnp.float32)
            return carry
        lax.fori_loop(0, CH, zrow, 0)

        def zchunk(i, carry):
            ch = sid + NS * i

            @pl.when(ch < ROW_CHUNKS)
            def _():
                pltpu.sync_copy(rows, acc.at[pl.ds(ch * CH, CH)])
            return carry
        lax.fori_loop(0, (ROW_CHUNKS + NS - 1) // NS, zchunk, 0)
        plsc.subcore_barrier()

        def run_edges(src_hbm, dst_hbm, table_hbm):
            ahead = NBUF - 2

            def issue(base, j):
                hg = pltpu.async_copy(
                    table_hbm.at[idx_sv.at[pl.ds(j * CH, CH)]],
                    bufs[j % NBUF], sems[j % NBUF])
                hd = pltpu.async_copy(
                    dst_hbm.at[pl.ds(base + j * CH, CH)],
                    dbufs[j % NBUF], dsems[j % NBUF])
                return hg, hd

            def blk(b, carry):
                base = sid * EDGE_PER_TILE + b * (IB * CH)
                pltpu.sync_copy(src_hbm.at[pl.ds(base, IB * CH)], idx_sv)
                hg, hd, hs = [], [], []
                for j in range(ahead):
                    g, d = issue(base, j)
                    hg.append(g)
                    hd.append(d)
                for j in range(IB):
                    if j >= 2:
                        hs[j - 2].wait()
                    if j + ahead < IB:
                        g, d = issue(base, j + ahead)
                        hg.append(g)
                        hd.append(d)
                    hg[j].wait()
                    hd[j].wait()
                    hs.append(pltpu.async_copy(
                        bufs[j % NBUF], acc.at[dbufs[j % NBUF]],
                        ssems[j % NBUF], add=True))
                hs[IB - 2].wait()
                hs[IB - 1].wait()
                return carry
            lax.fori_loop(0, NBLK, blk, 0)

        @pl.when(cid == 0)
        def _():
            run_edges(iu_s_hbm, iu_d_hbm, ti_hbm)

        @pl.when(cid == 1)
        def _():
            run_edges(ui_s_hbm, ui_d_hbm, tu_hbm)

        plsc.subcore_barrier()

        def write_out(res_hbm, out_hbm):
            def wchunk(i, carry):
                ch = sid + NS * i

                @pl.when(ch < ROW_CHUNKS)
                def _():
                    base = ch * CH
                    if add_residual:
                        pltpu.sync_copy(acc.at[pl.ds(base, CH)], rows)
                        pltpu.sync_copy(res_hbm.at[pl.ds(base, CH)], resb)

                        def arow(r, c2):
                            for j in range(F // L):
                                sl = pl.ds(j * L, L)
                                rows[r, sl] = rows[r, sl] + resb[r, sl]
                            return c2
                        lax.fori_loop(0, CH, arow, 0)
                        pltpu.sync_copy(rows, out_hbm.at[pl.ds(base, CH)])
                    else:
                        pltpu.sync_copy(acc.at[pl.ds(base, CH)],
                                        out_hbm.at[pl.ds(base, CH)])
                return carry
            lax.fori_loop(0, (ROW_CHUNKS + NS - 1) // NS, wchunk, 0)

        @pl.when(cid == 0)
        def _():
            write_out(ru_hbm, out_u_hbm)

        @pl.when(cid == 1)
        def _():
            write_out(ri_hbm, out_i_hbm)

    return k(table_u, table_i, ui_src, ui_dst, iu_src, iu_dst, res_u, res_i)


def kernel(x_user, x_item, edge_index_ui, edge_index_iu, time_user,
           batch_user, seed_time, W_enc_u, b_enc_u, W_enc_i, b_enc_i,
           W_time, b_time, W1_ui_l, W1_ui_r, b1_i, W1_iu_l, W1_iu_r, b1_u,
           W2_ui_l, W2_ui_r, b2_i, W2_iu_l, W2_iu_r, b2_u):
    ui_src = edge_index_ui[0].astype(jnp.int32)
    ui_dst = edge_index_ui[1].astype(jnp.int32)
    iu_src = edge_index_iu[0].astype(jnp.int32)
    iu_dst = edge_index_iu[1].astype(jnp.int32)
    batch_user = batch_user.astype(jnp.int32)

    g_u, g_i, r_u, r_i = _tc_stage1(
        x_user, x_item, batch_user, seed_time, time_user,
        W_enc_u, b_enc_u, W_enc_i, b_enc_i,
        W_time, b_time, W1_ui_l, W1_iu_l, W1_iu_r, b1_u, W1_ui_r, b1_i)

    msg_u, msg_i = _segsum_pair(g_u, g_i, ui_src, ui_dst, iu_src, iu_dst,
                                r_u, r_i, add_residual=False)

    g2_u, g2_i, r2_u, r2_i = _tc_stage2(
        msg_u, r_u, msg_i, r_i, W2_ui_l, W2_iu_l, W2_iu_r, b2_u,
        W2_ui_r, b2_i)

    u2, i2 = _segsum_pair(g2_u, g2_i, ui_src, ui_dst, iu_src, iu_dst,
                          r2_u, r2_i, add_residual=True)
    return (u2, i2)


# speedup vs baseline: 1.0878x; 1.0878x over previous
import functools

import numpy as np
import jax
import jax.numpy as jnp
from jax import lax
from jax.experimental import pallas as pl
from jax.experimental.pallas import tpu as pltpu
from jax.experimental.pallas import tpu_sc as plsc

N = 10000
F = 128
E_EDGES = 320000
NB = 1024

NC = 2
NS = 16
L = 16
NW = NC * NS

CH = 80
ROW_CHUNKS = N // CH
EDGE_PER_TILE = E_EDGES // NS
EDGE_CHUNKS = EDGE_PER_TILE // CH


_PE_DIV = np.exp(
    -np.floor(np.arange(F, dtype=np.float64) / 2.0)
    * (np.log(10000.0) / (F // 2))
).astype(np.float32)

_TC_BLK = 1000


def _tc_stage1(x_u, x_i, batch_user, seed_time, time_user,
               W_enc_u, b_enc_u, W_enc_i, b_enc_i,
               W_time, b_time, W1_ui_l, W1_iu_l, W1_iu_r, b1_u,
               W1_ui_r, b1_i):
    blk = _TC_BLK

    def body(xu_ref, xi_ref, bat_ref, seed_ref, tim_ref, pediv,
             weu, beu, wei, bei, wt, bt,
             wl_ui, wl_iu, wr_iu, bu, wr_ui, bi,
             gu_ref, gi_ref, ru_ref, ri_ref):
        dot = functools.partial(jnp.dot, preferred_element_type=jnp.float32)
        bid = lax.broadcasted_iota(jnp.int32, (blk, NB), 1)
        onehot = (bid == bat_ref[...]).astype(jnp.float32)
        rel = jnp.dot(onehot, seed_ref[...],
                      preferred_element_type=jnp.float32,
                      precision=lax.Precision.HIGHEST) - tim_ref[...]
        arg = rel * pediv[...]
        lane = lax.broadcasted_iota(jnp.int32, (blk, F), 1)
        pe = jnp.where(lane % 2 == 0, jnp.sin(arg), jnp.cos(arg))
        hu = dot(xu_ref[...], weu[...]) + beu[...] + dot(pe, wt[...]) + bt[...]
        hi = dot(xi_ref[...], wei[...]) + bei[...]
        gu_ref[...] = dot(hu, wl_ui[...])
        gi_ref[...] = dot(hi, wl_iu[...])
        ru_ref[...] = dot(hu, wr_iu[...]) + bu[...]
        ri_ref[...] = dot(hi, wr_ui[...]) + bi[...]

    row_spec = pl.BlockSpec((blk, F), lambda i: (i, 0))
    one_spec = pl.BlockSpec((blk, 1), lambda i: (i, 0))
    seed_spec = pl.BlockSpec((NB, 1), lambda i: (0, 0))
    w_spec = pl.BlockSpec((F, F), lambda i: (0, 0))
    b_spec = pl.BlockSpec((1, F), lambda i: (0, 0))
    out_sds = jax.ShapeDtypeStruct((N, F), jnp.float32)
    return pl.pallas_call(
        body,
        grid=(N // blk,),
        in_specs=[row_spec, row_spec, one_spec, seed_spec, one_spec, b_spec,
                  w_spec, b_spec, w_spec, b_spec, w_spec, b_spec,
                  w_spec, w_spec, w_spec, b_spec, w_spec, b_spec],
        out_specs=[row_spec, row_spec, row_spec, row_spec],
        out_shape=[out_sds, out_sds, out_sds, out_sds],
    )(x_u, x_i, batch_user.reshape(N, 1), seed_time.reshape(NB, 1),
      time_user.reshape(N, 1), jnp.asarray(_PE_DIV).reshape(1, F),
      W_enc_u, b_enc_u.reshape(1, F), W_enc_i, b_enc_i.reshape(1, F),
      W_time, b_time.reshape(1, F),
      W1_ui_l, W1_iu_l, W1_iu_r, b1_u.reshape(1, F),
      W1_ui_r, b1_i.reshape(1, F))


def _tc_stage2(msg_u, r_u, msg_i, r_i, W2_ui_l, W2_iu_l, W2_iu_r, b2_u,
               W2_ui_r, b2_i):
    blk = _TC_BLK

    def body(mu_ref, ru_ref, mi_ref, ri_ref,
             wl_ui, wl_iu, wr_iu, bu, wr_ui, bi,
             g2u_ref, g2i_ref, r2u_ref, r2i_ref):
        u1 = jnp.maximum(mu_ref[...] + ru_ref[...], 0.0)
        i1 = jnp.maximum(mi_ref[...] + ri_ref[...], 0.0)
        dot = functools.partial(jnp.dot, preferred_element_type=jnp.float32)
        g2u_ref[...] = dot(u1, wl_ui[...])
        g2i_ref[...] = dot(i1, wl_iu[...])
        r2u_ref[...] = dot(u1, wr_iu[...]) + bu[...]
        r2i_ref[...] = dot(i1, wr_ui[...]) + bi[...]

    row_spec = pl.BlockSpec((blk, F), lambda i: (i, 0))
    w_spec = pl.BlockSpec((F, F), lambda i: (0, 0))
    b_spec = pl.BlockSpec((1, F), lambda i: (0, 0))
    out_sds = jax.ShapeDtypeStruct((N, F), jnp.float32)
    return pl.pallas_call(
        body,
        grid=(N // blk,),
        in_specs=[row_spec, row_spec, row_spec, row_spec,
                  w_spec, w_spec, w_spec, b_spec, w_spec, b_spec],
        out_specs=[row_spec, row_spec, row_spec, row_spec],
        out_shape=[out_sds, out_sds, out_sds, out_sds],
    )(msg_u, r_u, msg_i, r_i,
      W2_ui_l, W2_iu_l, W2_iu_r, b2_u.reshape(1, F),
      W2_ui_r, b2_i.reshape(1, F))


IB = 125
NBLK = EDGE_CHUNKS // IB
NBUF = 3


def _segsum_pair(table_u, table_i, ui_src, ui_dst, iu_src, iu_dst,
                 res_u, res_i, add_residual):
    mesh = plsc.VectorSubcoreMesh(core_axis_name="c", subcore_axis_name="s")

    @functools.partial(
        pl.kernel, mesh=mesh,
        out_type=(jax.ShapeDtypeStruct((N, F), jnp.float32),
                  jax.ShapeDtypeStruct((N, F), jnp.float32)),
        scratch_types=[
            pltpu.VMEM_SHARED((N, F), jnp.float32),
            pltpu.VMEM((IB * CH,), jnp.int32),
        ]
        + [pltpu.VMEM((CH,), jnp.int32) for _ in range(NBUF)]
        + [pltpu.VMEM((CH, F), jnp.float32) for _ in range(NBUF)]
        + [pltpu.SemaphoreType.DMA for _ in range(2 * NBUF)],
    )
    def k(tu_hbm, ti_hbm, ui_s_hbm, ui_d_hbm, iu_s_hbm, iu_d_hbm,
          ru_hbm, ri_hbm, out_u_hbm, out_i_hbm,
          acc, idx_sv, *bufs_and_sems):
        cid = lax.axis_index("c")
        sid = lax.axis_index("s")
        dbufs = bufs_and_sems[0:NBUF]
        bufs = bufs_and_sems[NBUF:2 * NBUF]
        sems = bufs_and_sems[2 * NBUF:3 * NBUF]
        dsems = bufs_and_sems[3 * NBUF:4 * NBUF]
        rows = bufs[0]
        resb = bufs[1]

        def zrow(r, carry):
            for j in range(F // L):
                rows[r, pl.ds(j * L, L)] = jnp.zeros((L,), jnp.float32)
            return carry
        lax.fori_loop(0, CH, zrow, 0)

        def zchunk(i, carry):
            ch = sid + NS * i

            @pl.when(ch < ROW_CHUNKS)
            def _():
                pltpu.sync_copy(rows, acc.at[pl.ds(ch * CH, CH)])
            return carry
        lax.fori_loop(0, (ROW_CHUNKS + NS - 1) // NS, zchunk, 0)
        plsc.subcore_barrier()

        def run_edges(src_hbm, dst_hbm, table_hbm):
            ahead = NBUF - 1

            def issue(base, j):
                hg = pltpu.async_copy(
                    table_hbm.at[idx_sv.at[pl.ds(j * CH, CH)]],
                    bufs[j % NBUF], sems[j % NBUF])
                hd = pltpu.async_copy(
                    dst_hbm.at[pl.ds(base + j * CH, CH)],
                    dbufs[j % NBUF], dsems[j % NBUF])
                return hg, hd

            def blk(b, carry):
                base = sid * EDGE_PER_TILE + b * (IB * CH)
                pltpu.sync_copy(src_hbm.at[pl.ds(base, IB * CH)], idx_sv)
                hg, hd = [], []
                for j in range(ahead):
                    g, d = issue(base, j)
                    hg.append(g)
                    hd.append(d)
                for j in range(IB):
                    if j + ahead < IB:
                        g, d = issue(base, j + ahead)
                        hg.append(g)
                        hd.append(d)
                    hg[j].wait()
                    hd[j].wait()
                    pltpu.sync_copy(bufs[j % NBUF],
                                    acc.at[dbufs[j % NBUF]], add=True)
                return carry
            lax.fori_loop(0, NBLK, blk, 0)

        @pl.when(cid == 0)
        def _():
            run_edges(iu_s_hbm, iu_d_hbm, ti_hbm)

        @pl.when(cid == 1)
        def _():
            run_edges(ui_s_hbm, ui_d_hbm, tu_hbm)

        plsc.subcore_barrier()

        def write_out(res_hbm, out_hbm):
            def wchunk(i, carry):
                ch = sid + NS * i

                @pl.when(ch < ROW_CHUNKS)
                def _():
                    base = ch * CH
                    if add_residual:
                        pltpu.sync_copy(acc.at[pl.ds(base, CH)], rows)
                        pltpu.sync_copy(res_hbm.at[pl.ds(base, CH)], resb)

                        def arow(r, c2):
                            for j in range(F // L):
                                sl = pl.ds(j * L, L)
                                rows[r, sl] = rows[r, sl] + resb[r, sl]
                            return c2
                        lax.fori_loop(0, CH, arow, 0)
                        pltpu.sync_copy(rows, out_hbm.at[pl.ds(base, CH)])
                    else:
                        pltpu.sync_copy(acc.at[pl.ds(base, CH)],
                                        out_hbm.at[pl.ds(base, CH)])
                return carry
            lax.fori_loop(0, (ROW_CHUNKS + NS - 1) // NS, wchunk, 0)

        @pl.when(cid == 0)
        def _():
            write_out(ru_hbm, out_u_hbm)

        @pl.when(cid == 1)
        def _():
            write_out(ri_hbm, out_i_hbm)

    return k(table_u, table_i, ui_src, ui_dst, iu_src, iu_dst, res_u, res_i)


def kernel(x_user, x_item, edge_index_ui, edge_index_iu, time_user,
           batch_user, seed_time, W_enc_u, b_enc_u, W_enc_i, b_enc_i,
           W_time, b_time, W1_ui_l, W1_ui_r, b1_i, W1_iu_l, W1_iu_r, b1_u,
           W2_ui_l, W2_ui_r, b2_i, W2_iu_l, W2_iu_r, b2_u):
    ui_src = edge_index_ui[0].astype(jnp.int32)
    ui_dst = edge_index_ui[1].astype(jnp.int32)
    iu_src = edge_index_iu[0].astype(jnp.int32)
    iu_dst = edge_index_iu[1].astype(jnp.int32)
    batch_user = batch_user.astype(jnp.int32)

    g_u, g_i, r_u, r_i = _tc_stage1(
        x_user, x_item, batch_user, seed_time, time_user,
        W_enc_u, b_enc_u, W_enc_i, b_enc_i,
        W_time, b_time, W1_ui_l, W1_iu_l, W1_iu_r, b1_u, W1_ui_r, b1_i)

    msg_u, msg_i = _segsum_pair(g_u, g_i, ui_src, ui_dst, iu_src, iu_dst,
                                r_u, r_i, add_residual=False)

    g2_u, g2_i, r2_u, r2_i = _tc_stage2(
        msg_u, r_u, msg_i, r_i, W2_ui_l, W2_iu_l, W2_iu_r, b2_u,
        W2_ui_r, b2_i)

    u2, i2 = _segsum_pair(g2_u, g2_i, ui_src, ui_dst, iu_src, iu_dst,
                          r2_u, r2_i, add_residual=True)
    return (u2, i2)
